# baseline (device time: 19536 ns/iter reference)
import jax
import jax.numpy as jnp
from jax import lax
from jax.experimental import pallas as pl
from jax.experimental.pallas import tpu as pltpu

N_DEV = 8
N_WQ = 4
_GELU_C = 0.7978845608028654


def _gelu_bf16(y):
    y = 0.5 * y * (1.0 + jnp.tanh(_GELU_C * (y + 0.044715 * y * y * y)))
    return y.astype(jnp.bfloat16)


def kernel(x, w_mat):
    m_per, k = x.shape
    _, n = w_mat.shape
    blk = n // N_DEV
    kq = k // N_WQ

    def body(x_hbm, w_hbm, out_ref, xv_ref, w_ref, y_ref, part_ref, own_sem,
             x_sem, w_sems, send_sems, recv_sems):
        my_i = lax.axis_index("i")

        x_cp = pltpu.make_async_copy(x_hbm, xv_ref, x_sem)
        x_cp.start()
        w_cps = []
        for q in range(N_WQ):
            cp = pltpu.make_async_copy(
                w_hbm.at[pl.ds(q * kq, kq), :],
                w_ref.at[pl.ds(q * kq, kq), :],
                w_sems.at[q],
            )
            cp.start()
            w_cps.append(cp)

        barrier_sem = pltpu.get_barrier_semaphore()
        for j in range(1, N_DEV):
            pl.semaphore_signal(
                barrier_sem, inc=1,
                device_id=((my_i + j) % N_DEV,),
                device_id_type=pl.DeviceIdType.MESH,
            )
        pl.semaphore_wait(barrier_sem, N_DEV - 1)

        x_cp.wait()

        for s in range(N_WQ - 1):
            w_cps[s].wait()
            for j in range(N_DEV):
                p = (my_i + j) % N_DEV
                part = jnp.dot(
                    xv_ref[:, pl.ds(s * kq, kq)],
                    w_ref[pl.ds(s * kq, kq), pl.ds(p * blk, blk)],
                    preferred_element_type=jnp.float32)
                if s == 0:
                    part_ref[j, :, :] = part
                else:
                    part_ref[j, :, :] = part_ref[j, :, :] + part

        w_cps[N_WQ - 1].wait()
        sq = (N_WQ - 1) * kq

        sends = []
        for j in range(1, N_DEV):
            p = (my_i + j) % N_DEV
            y_ref[j, :, :] = _gelu_bf16(
                part_ref[j, :, :]
                + jnp.dot(xv_ref[:, sq:], w_ref[sq:, pl.ds(p * blk, blk)],
                          preferred_element_type=jnp.float32)
            )
            rdma = pltpu.make_async_remote_copy(
                src_ref=y_ref.at[j],
                dst_ref=out_ref.at[pl.ds(my_i * m_per, m_per), :],
                send_sem=send_sems.at[j],
                recv_sem=recv_sems.at[j],
                device_id=(p,),
                device_id_type=pl.DeviceIdType.MESH,
            )
            rdma.start()
            sends.append(rdma)

        y_ref[0, :, :] = _gelu_bf16(
            part_ref[0, :, :]
            + jnp.dot(xv_ref[:, sq:], w_ref[sq:, pl.ds(my_i * blk, blk)],
                      preferred_element_type=jnp.float32)
        )
        own_cp = pltpu.make_async_copy(
            y_ref.at[0],
            out_ref.at[pl.ds(my_i * m_per, m_per), :],
            own_sem,
        )
        own_cp.start()

        for j in range(1, N_DEV):
            s = (my_i - j) % N_DEV
            recv = pltpu.make_async_remote_copy(
                src_ref=y_ref.at[j],
                dst_ref=out_ref.at[pl.ds(s * m_per, m_per), :],
                send_sem=send_sems.at[j],
                recv_sem=recv_sems.at[j],
                device_id=(s,),
                device_id_type=pl.DeviceIdType.MESH,
            )
            recv.wait_recv()

        own_cp.wait()
        for rdma in sends:
            rdma.wait_send()

    x = pltpu.with_memory_space_constraint(x, pltpu.MemorySpace.HBM)
    w_mat = pltpu.with_memory_space_constraint(w_mat, pltpu.MemorySpace.HBM)
    out_shape = jax.ShapeDtypeStruct((N_DEV * m_per, blk), jnp.bfloat16)
    return pl.pallas_call(
        body,
        out_shape=out_shape,
        in_specs=[
            pl.BlockSpec(memory_space=pltpu.MemorySpace.HBM),
            pl.BlockSpec(memory_space=pltpu.MemorySpace.HBM),
        ],
        out_specs=pl.BlockSpec(memory_space=pltpu.MemorySpace.HBM),
        scratch_shapes=[
            pltpu.VMEM((m_per, k), jnp.float32),
            pltpu.VMEM((k, n), jnp.float32),
            pltpu.VMEM((N_DEV, m_per, blk), jnp.bfloat16),
            pltpu.VMEM((N_DEV, m_per, blk), jnp.float32),
            pltpu.SemaphoreType.DMA,
            pltpu.SemaphoreType.DMA,
            pltpu.SemaphoreType.DMA((N_WQ,)),
            pltpu.SemaphoreType.DMA((N_DEV,)),
            pltpu.SemaphoreType.DMA((N_DEV,)),
        ],
        compiler_params=pltpu.CompilerParams(collective_id=0),
    )(x, w_mat)


# device time: 18940 ns/iter; 1.0315x vs baseline; 1.0315x over previous
import jax
import jax.numpy as jnp
from jax import lax
from jax.experimental import pallas as pl
from jax.experimental.pallas import tpu as pltpu

N_DEV = 8
N_WQ = 4
_GELU_C = 0.7978845608028654


def _gelu_bf16(y):
    y = 0.5 * y * (1.0 + jnp.tanh(_GELU_C * (y + 0.044715 * y * y * y)))
    return y.astype(jnp.bfloat16)


def kernel(x, w_mat):
    m_per, k = x.shape
    _, n = w_mat.shape
    blk = n // N_DEV
    kq = k // N_WQ

    def body(x_hbm, w_hbm, out_ref, xv_ref, w_ref, y_ref, part_ref, own_sem,
             x_sem, w_sems, send_sems, recv_sems):
        my_i = lax.axis_index("i")

        x_cp = pltpu.make_async_copy(x_hbm, xv_ref, x_sem)
        x_cp.start()
        w_cps = []
        for q in range(N_WQ):
            cp = pltpu.make_async_copy(
                w_hbm.at[pl.ds(q * kq, kq), :],
                w_ref.at[pl.ds(q * kq, kq), :],
                w_sems.at[q],
            )
            cp.start()
            w_cps.append(cp)

        barrier_sem = pltpu.get_barrier_semaphore()
        for j in range(1, N_DEV):
            pl.semaphore_signal(
                barrier_sem, inc=1,
                device_id=((my_i + j) % N_DEV,),
                device_id_type=pl.DeviceIdType.MESH,
            )
        pl.semaphore_wait(barrier_sem, N_DEV - 1)

        x_cp.wait()
        kh = k // 2
        w_cps[0].wait()
        w_cps[1].wait()

        for j in range(N_DEV):
            p = (my_i + j) % N_DEV
            part_ref[j, :, :] = jnp.dot(
                xv_ref[:, :kh], w_ref[:kh, pl.ds(p * blk, blk)],
                preferred_element_type=jnp.float32)

        w_cps[2].wait()
        w_cps[3].wait()

        sends = []
        for j in range(1, N_DEV):
            p = (my_i + j) % N_DEV
            y_ref[j, :, :] = _gelu_bf16(
                part_ref[j, :, :]
                + jnp.dot(xv_ref[:, kh:], w_ref[kh:, pl.ds(p * blk, blk)],
                          preferred_element_type=jnp.float32)
            )
            rdma = pltpu.make_async_remote_copy(
                src_ref=y_ref.at[j],
                dst_ref=out_ref.at[pl.ds(my_i * m_per, m_per), :],
                send_sem=send_sems.at[j],
                recv_sem=recv_sems.at[j],
                device_id=(p,),
                device_id_type=pl.DeviceIdType.MESH,
            )
            rdma.start()
            sends.append(rdma)

        y_ref[0, :, :] = _gelu_bf16(
            part_ref[0, :, :]
            + jnp.dot(xv_ref[:, kh:], w_ref[kh:, pl.ds(my_i * blk, blk)],
                      preferred_element_type=jnp.float32)
        )
        own_cp = pltpu.make_async_copy(
            y_ref.at[0],
            out_ref.at[pl.ds(my_i * m_per, m_per), :],
            own_sem,
        )
        own_cp.start()

        for j in range(1, N_DEV):
            s = (my_i - j) % N_DEV
            recv = pltpu.make_async_remote_copy(
                src_ref=y_ref.at[j],
                dst_ref=out_ref.at[pl.ds(s * m_per, m_per), :],
                send_sem=send_sems.at[j],
                recv_sem=recv_sems.at[j],
                device_id=(s,),
                device_id_type=pl.DeviceIdType.MESH,
            )
            recv.wait_recv()

        own_cp.wait()
        for rdma in sends:
            rdma.wait_send()

    x = pltpu.with_memory_space_constraint(x, pltpu.MemorySpace.HBM)
    w_mat = pltpu.with_memory_space_constraint(w_mat, pltpu.MemorySpace.HBM)
    out_shape = jax.ShapeDtypeStruct((N_DEV * m_per, blk), jnp.bfloat16)
    return pl.pallas_call(
        body,
        out_shape=out_shape,
        in_specs=[
            pl.BlockSpec(memory_space=pltpu.MemorySpace.HBM),
            pl.BlockSpec(memory_space=pltpu.MemorySpace.HBM),
        ],
        out_specs=pl.BlockSpec(memory_space=pltpu.MemorySpace.HBM),
        scratch_shapes=[
            pltpu.VMEM((m_per, k), jnp.float32),
            pltpu.VMEM((k, n), jnp.float32),
            pltpu.VMEM((N_DEV, m_per, blk), jnp.bfloat16),
            pltpu.VMEM((N_DEV, m_per, blk), jnp.float32),
            pltpu.SemaphoreType.DMA,
            pltpu.SemaphoreType.DMA,
            pltpu.SemaphoreType.DMA((N_WQ,)),
            pltpu.SemaphoreType.DMA((N_DEV,)),
            pltpu.SemaphoreType.DMA((N_DEV,)),
        ],
        compiler_params=pltpu.CompilerParams(collective_id=0),
    )(x, w_mat)
